# baseline (speedup 1.0000x reference)
import functools

import jax
import jax.numpy as jnp
from jax import lax
from jax.experimental import pallas as pl
from jax.experimental.pallas import tpu as pltpu
from jax.experimental.pallas import tpu_sc as plsc

_N = 10000
_E = 320000
_IN = 128
_HID = 64
_HEADS = 8
_G = 32
_BR = 1000
_GRID = _N // _BR
_EPAD = 360448
_GROUPS = _EPAD // 128
_GT1 = _GROUPS // 16
_GT2 = _GROUPS // 32
_NP16 = _N + 16
_SZ = 632
_ZLAST = _NP16 - 15 * _SZ
_OLAST = _N - 15 * _SZ

_f32 = jnp.float32
_i32 = jnp.int32


def _dense1_body(x_ref, w1_ref, as_ref, ad_ref, *outs):
    h_refs = outs[:_HEADS]
    s_refs = outs[_HEADS:2 * _HEADS]
    d_refs = outs[2 * _HEADS:]
    xb = x_ref[...]
    for h in range(_HEADS):
        wh = w1_ref[:, h * _HID:(h + 1) * _HID]
        hh = jnp.dot(xb, wh, preferred_element_type=_f32)
        h_refs[h][...] = hh
        s_refs[h][...] = jnp.sum(hh * as_ref[h:h + 1, :], axis=1,
                                 keepdims=True)
        d_refs[h][...] = jnp.sum(hh * ad_ref[h:h + 1, :], axis=1,
                                 keepdims=True)


def _dense1(x, W1, att_src1, att_dst1):
    outs = ([jax.ShapeDtypeStruct((_N, _HID), _f32)] * _HEADS
            + [jax.ShapeDtypeStruct((_N, 1), _f32)] * (2 * _HEADS))
    out_specs = ([pl.BlockSpec((_BR, _HID), lambda i: (i, 0))] * _HEADS
                 + [pl.BlockSpec((_BR, 1), lambda i: (i, 0))] * (2 * _HEADS))
    return pl.pallas_call(
        _dense1_body,
        grid=(_GRID,),
        in_specs=[
            pl.BlockSpec((_BR, _IN), lambda i: (i, 0)),
            pl.BlockSpec((_IN, _HEADS * _HID), lambda i: (0, 0)),
            pl.BlockSpec((_HEADS, _HID), lambda i: (0, 0)),
            pl.BlockSpec((_HEADS, _HID), lambda i: (0, 0)),
        ],
        out_specs=out_specs,
        out_shape=outs,
    )(x, W1, att_src1, att_dst1)


_SB = 2


def _sc_group_body(h_ref, asrc_ref, adst_ref, src_v, dst_v, avs_v, avd_v,
                   rows_v, wlin_v, wv2d_v, acc_sh, den_sh, sem, sem2):
    lanes = lax.broadcasted_iota(_i32, (16,), 0)

    def body(s, carry):
        gathers = []
        for b in range(_SB):
            g = s * _SB + b
            sidx = src_v.at[g]
            didx = dst_v.at[g]
            gathers.append(pltpu.async_copy(h_ref.at[sidx],
                                            rows_v.at[b], sem))
            gathers.append(pltpu.async_copy(asrc_ref.at[sidx],
                                            avs_v.at[b], sem))
            gathers.append(pltpu.async_copy(adst_ref.at[didx],
                                            avd_v.at[b], sem))
        for cp in gathers:
            cp.wait()

        def wphase(q, c4):
            b = q // 8
            sl = pl.ds((q % 8) * 16, 16)
            e = avs_v[b, sl] + avd_v[b, sl]
            e = jnp.maximum(e, e * 0.2)
            wlin_v[b, sl] = jnp.exp(e)
            return c4

        lax.fori_loop(0, _SB * 8, wphase, 0)

        def scale(k, c4):
            b = k // 128
            r = k % 128
            w16 = wlin_v[b, pl.ds((r // 16) * 16, 16)]
            ws = jnp.sum(jnp.where(lanes == r % 16, w16, 0.0))
            spl = ws * jnp.ones((16,), _f32)
            wv2d_v[b, r, :] = spl
            for cc in range(4):
                s2 = pl.ds(cc * 16, 16)
                rows_v[b, r, s2] = rows_v[b, r, s2] * spl
            return c4

        lax.fori_loop(0, _SB * 128, scale, 0)
        scats = []
        for b in range(_SB):
            didx = dst_v.at[s * _SB + b]
            scats.append(pltpu.async_copy(rows_v.at[b], acc_sh.at[didx],
                                          sem2, add=True))
            scats.append(pltpu.async_copy(wv2d_v.at[b], den_sh.at[didx],
                                          sem2, add=True))
        for cp in scats:
            cp.wait()
        return carry
    return body


def _zero_stripes(sid, z64, z16, acc_sh, den_sh):
    zs = pl.multiple_of(sid * _SZ, 8)

    @pl.when(sid < 15)
    def _():
        pltpu.sync_copy(z64, acc_sh.at[pl.ds(zs, _SZ)])
        pltpu.sync_copy(z16, den_sh.at[pl.ds(zs, _SZ)])

    @pl.when(sid == 15)
    def _():
        pltpu.sync_copy(z64.at[pl.ds(0, _ZLAST)],
                        acc_sh.at[pl.ds(15 * _SZ, _ZLAST)])
        pltpu.sync_copy(z16.at[pl.ds(0, _ZLAST)],
                        den_sh.at[pl.ds(15 * _SZ, _ZLAST)])


def _out_stripes(sid, acc_sh, den_sh, num_slot, den_slot):
    os_ = pl.multiple_of(sid * _SZ, 8)

    @pl.when(sid < 15)
    def _():
        pltpu.sync_copy(acc_sh.at[pl.ds(os_, _SZ)],
                        num_slot.at[pl.ds(os_, _SZ)])
        pltpu.sync_copy(den_sh.at[pl.ds(os_, _SZ)],
                        den_slot.at[pl.ds(os_, _SZ)])

    @pl.when(sid == 15)
    def _():
        pltpu.sync_copy(acc_sh.at[pl.ds(15 * _SZ, _OLAST)],
                        num_slot.at[pl.ds(15 * _SZ, _OLAST)])
        pltpu.sync_copy(den_sh.at[pl.ds(15 * _SZ, _OLAST)],
                        den_slot.at[pl.ds(15 * _SZ, _OLAST)])


def _make_edge1():
    mesh = plsc.VectorSubcoreMesh(core_axis_name="c", subcore_axis_name="s")

    @functools.partial(
        pl.kernel, mesh=mesh,
        out_type=[jax.ShapeDtypeStruct((_HEADS, _N, _HID), _f32),
                  jax.ShapeDtypeStruct((_HEADS, _N, 16), _f32)],
        scratch_types=[
            pltpu.VMEM_SHARED((_NP16, _HID), _f32),
            pltpu.VMEM_SHARED((_NP16, 16), _f32),
            pltpu.VMEM((_GT1, 128), _i32),
            pltpu.VMEM((_GT1, 128), _i32),
            pltpu.VMEM((_SB, 128), _f32),
            pltpu.VMEM((_SB, 128), _f32),
            pltpu.VMEM((_SB, 128, _HID), _f32),
            pltpu.VMEM((_SB, 128), _f32),
            pltpu.VMEM((_SB, 128, 16), _f32),
            pltpu.SemaphoreType.DMA,
            pltpu.SemaphoreType.DMA,
        ],
        compiler_params=pltpu.CompilerParams(use_tc_tiling_on_sc=False, needs_layout_passes=False),
    )
    def edge1(src3, dst3, asrc, adst, z64, z16,
              h0, h1, h2, h3, h4, h5, h6, h7,
              num_out, den_out,
              acc_sh, den_sh, src_v, dst_v, avs_v, avd_v,
              rows_v, wlin_v, wv2d_v, sem, sem2):
        hs = (h0, h1, h2, h3, h4, h5, h6, h7)
        core = lax.axis_index("c")
        sid = lax.axis_index("s")
        gb = sid * _GT1
        pltpu.sync_copy(src3.at[pl.ds(gb, _GT1)], src_v)
        pltpu.sync_copy(dst3.at[pl.ds(gb, _GT1)], dst_v)
        for hp in range(_HEADS // 2):
            _zero_stripes(sid, z64, z16, acc_sh, den_sh)
            plsc.subcore_barrier()
            for c in range(2):
                head = c * (_HEADS // 2) + hp

                @pl.when(core == c)
                def _():
                    lax.fori_loop(
                        0, _GT1 // _SB,
                        _sc_group_body(hs[head], asrc.at[head],
                                       adst.at[head], src_v, dst_v,
                                       avs_v, avd_v, rows_v, wlin_v,
                                       wv2d_v, acc_sh, den_sh, sem, sem2),
                        0)
            plsc.subcore_barrier()
            for c in range(2):
                head = c * (_HEADS // 2) + hp

                @pl.when(core == c)
                def _():
                    _out_stripes(sid, acc_sh, den_sh,
                                 num_out.at[head], den_out.at[head])
            plsc.subcore_barrier()

    return edge1


def _make_edge2():
    mesh = plsc.VectorSubcoreMesh(core_axis_name="c", subcore_axis_name="s")

    @functools.partial(
        pl.kernel, mesh=mesh,
        out_type=[jax.ShapeDtypeStruct((2, _N, _HID), _f32),
                  jax.ShapeDtypeStruct((2, _N, 16), _f32)],
        scratch_types=[
            pltpu.VMEM_SHARED((_NP16, _HID), _f32),
            pltpu.VMEM_SHARED((_NP16, 16), _f32),
            pltpu.VMEM((_GT2, 128), _i32),
            pltpu.VMEM((_GT2, 128), _i32),
            pltpu.VMEM((_SB, 128), _f32),
            pltpu.VMEM((_SB, 128), _f32),
            pltpu.VMEM((_SB, 128, _HID), _f32),
            pltpu.VMEM((_SB, 128), _f32),
            pltpu.VMEM((_SB, 128, 16), _f32),
            pltpu.SemaphoreType.DMA,
            pltpu.SemaphoreType.DMA,
        ],
        compiler_params=pltpu.CompilerParams(use_tc_tiling_on_sc=False, needs_layout_passes=False),
    )
    def edge2(src3, dst3, a2, z64, z16, hfeat,
              num_out, den_out,
              acc_sh, den_sh, src_v, dst_v, avs_v, avd_v,
              rows_v, wlin_v, wv2d_v, sem, sem2):
        core = lax.axis_index("c")
        sid = lax.axis_index("s")
        for c in range(2):
            @pl.when(core == c)
            def _():
                gb = c * (_GROUPS // 2) + sid * _GT2
                pltpu.sync_copy(src3.at[pl.ds(gb, _GT2)], src_v)
                pltpu.sync_copy(dst3.at[pl.ds(gb, _GT2)], dst_v)
        _zero_stripes(sid, z64, z16, acc_sh, den_sh)
        plsc.subcore_barrier()
        lax.fori_loop(
            0, _GT2 // _SB,
            _sc_group_body(hfeat, a2.at[0], a2.at[1], src_v, dst_v,
                           avs_v, avd_v, rows_v, wlin_v, wv2d_v,
                           acc_sh, den_sh, sem, sem2),
            0)
        plsc.subcore_barrier()
        for c in range(2):
            @pl.when(core == c)
            def _():
                _out_stripes(sid, acc_sh, den_sh,
                             num_out.at[c], den_out.at[c])

    return edge2


def _finish1_body(num_ref, den_ref, b1_ref, w2_ref, as2_ref, ad2_ref,
                  h2_ref, a2s_ref, a2d_ref):
    acc = jnp.zeros((_BR, _HID), _f32)
    for h in range(_HEADS):
        seg = num_ref[h] / den_ref[h, :, 0:1]
        seg = jnp.maximum(seg + b1_ref[h:h + 1, :], 0.0)
        acc = acc + jnp.dot(seg, w2_ref[h], preferred_element_type=_f32)
    h2_ref[...] = acc
    a2s_ref[...] = jnp.sum(acc * as2_ref[...], axis=1, keepdims=True)
    a2d_ref[...] = jnp.sum(acc * ad2_ref[...], axis=1, keepdims=True)


def _finish1(num1, den1, b1r, W2r, att_src2, att_dst2):
    return pl.pallas_call(
        _finish1_body,
        grid=(_GRID,),
        in_specs=[
            pl.BlockSpec((_HEADS, _BR, _HID), lambda i: (0, i, 0)),
            pl.BlockSpec((_HEADS, _BR, 16), lambda i: (0, i, 0)),
            pl.BlockSpec((_HEADS, _HID), lambda i: (0, 0)),
            pl.BlockSpec((_HEADS, _HID, _HID), lambda i: (0, 0, 0)),
            pl.BlockSpec((1, _HID), lambda i: (0, 0)),
            pl.BlockSpec((1, _HID), lambda i: (0, 0)),
        ],
        out_specs=[
            pl.BlockSpec((_BR, _HID), lambda i: (i, 0)),
            pl.BlockSpec((_BR, 1), lambda i: (i, 0)),
            pl.BlockSpec((_BR, 1), lambda i: (i, 0)),
        ],
        out_shape=[jax.ShapeDtypeStruct((_N, _HID), _f32),
                   jax.ShapeDtypeStruct((_N, 1), _f32),
                   jax.ShapeDtypeStruct((_N, 1), _f32)],
    )(num1, den1, b1r, W2r, att_src2, att_dst2)


def _finish2_body(num_ref, den_ref, batch_ref, b2_ref, wfc_ref, out_ref):
    bid = batch_ref[...]
    oh = (lax.broadcasted_iota(_i32, (_G, _N), 0) == bid).astype(_f32)
    hsum = num_ref[0] + num_ref[1]
    dsum = den_ref[0, :, 0:1] + den_ref[1, :, 0:1]
    h2o = jnp.maximum(hsum / dsum + b2_ref[0:1, :], 0.0)
    ps = lax.dot_general(oh, h2o, (((1,), (0,)), ((), ())),
                         preferred_element_type=_f32)
    cnt = jnp.sum(oh, axis=1, keepdims=True)
    pooled = ps / jnp.maximum(cnt, 1.0)
    out_ref[...] = jnp.dot(pooled, wfc_ref[...], preferred_element_type=_f32)


def _finish2(num2, den2, batch2, b2r, Wfcp):
    return pl.pallas_call(
        _finish2_body,
        out_shape=jax.ShapeDtypeStruct((_G, 128), _f32),
    )(num2, den2, batch2, b2r, Wfcp)


_edge1_call = _make_edge1()
_edge2_call = _make_edge2()


def kernel(x, edge_index, batch, W1, att_src1, att_dst1, b1,
           W2, att_src2, att_dst2, b2, Wfc, bfc):
    loop = jnp.arange(_N, dtype=_i32)
    src = jnp.concatenate([edge_index[0].astype(_i32), loop])
    dst = jnp.concatenate([edge_index[1].astype(_i32), loop])
    padn = _EPAD - (_E + _N)
    src = jnp.concatenate([src, jnp.zeros((padn,), _i32)])
    dst = jnp.concatenate([dst, jnp.full((padn,), _N, _i32)])
    src3 = src.reshape(_GROUPS, 128)
    dst3 = dst.reshape(_GROUPS, 128)
    z64 = jnp.zeros((_SZ, _HID), _f32)
    z16 = jnp.zeros((_SZ, 16), _f32)

    d1 = _dense1(x, W1, att_src1, att_dst1)
    hs = d1[:_HEADS]
    asrc1 = jnp.concatenate([r.reshape(1, _N) for r in d1[_HEADS:2 * _HEADS]])
    adst1 = jnp.concatenate([r.reshape(1, _N) for r in d1[2 * _HEADS:]])

    num1, den1 = _edge1_call(src3, dst3, asrc1, adst1, z64, z16, *hs)

    h2f, a2s, a2d = _finish1(num1, den1, b1.reshape(_HEADS, _HID),
                             W2.reshape(_HEADS, _HID, _HID),
                             att_src2, att_dst2)
    a2 = jnp.concatenate([a2s.reshape(1, _N), a2d.reshape(1, _N)])

    num2, den2 = _edge2_call(src3, dst3, a2, z64, z16, h2f)

    batch2 = batch.astype(_i32).reshape(1, _N)
    b2r = b2.reshape(1, _HID)
    Wfcp = jnp.concatenate([Wfc, jnp.zeros((_HID, 127), _f32)], axis=1)
    fcout = _finish2(num2, den2, batch2, b2r, Wfcp)
    return fcout[:, :1] + bfc


# speedup vs baseline: 1.1680x; 1.1680x over previous
import functools

import jax
import jax.numpy as jnp
from jax import lax
from jax.experimental import pallas as pl
from jax.experimental.pallas import tpu as pltpu
from jax.experimental.pallas import tpu_sc as plsc

_N = 10000
_E = 320000
_IN = 128
_HID = 64
_HEADS = 8
_G = 32
_BR = 1000
_GRID = _N // _BR
_EPAD = 360448
_GROUPS = _EPAD // 128
_GT1 = _GROUPS // 16
_GT2 = _GROUPS // 32
_NP16 = _N + 16
_SZ = 632
_ZLAST = _NP16 - 15 * _SZ
_OLAST = _N - 15 * _SZ

_f32 = jnp.float32
_i32 = jnp.int32


def _dense1_body(x_ref, w1_ref, as_ref, ad_ref, *outs):
    h_refs = outs[:_HEADS]
    s_refs = outs[_HEADS:2 * _HEADS]
    d_refs = outs[2 * _HEADS:]
    xb = x_ref[...]
    for h in range(_HEADS):
        wh = w1_ref[:, h * _HID:(h + 1) * _HID]
        hh = jnp.dot(xb, wh, preferred_element_type=_f32)
        h_refs[h][...] = hh
        s_refs[h][...] = jnp.sum(hh * as_ref[h:h + 1, :], axis=1,
                                 keepdims=True)
        d_refs[h][...] = jnp.sum(hh * ad_ref[h:h + 1, :], axis=1,
                                 keepdims=True)


def _dense1(x, W1, att_src1, att_dst1):
    outs = ([jax.ShapeDtypeStruct((_N, _HID), _f32)] * _HEADS
            + [jax.ShapeDtypeStruct((_N, 1), _f32)] * (2 * _HEADS))
    out_specs = ([pl.BlockSpec((_BR, _HID), lambda i: (i, 0))] * _HEADS
                 + [pl.BlockSpec((_BR, 1), lambda i: (i, 0))] * (2 * _HEADS))
    return pl.pallas_call(
        _dense1_body,
        grid=(_GRID,),
        in_specs=[
            pl.BlockSpec((_BR, _IN), lambda i: (i, 0)),
            pl.BlockSpec((_IN, _HEADS * _HID), lambda i: (0, 0)),
            pl.BlockSpec((_HEADS, _HID), lambda i: (0, 0)),
            pl.BlockSpec((_HEADS, _HID), lambda i: (0, 0)),
        ],
        out_specs=out_specs,
        out_shape=outs,
    )(x, W1, att_src1, att_dst1)


def _sc_group_body(h_ref, asrc_ref, adst_ref, src_v, dst_v, avs_v, avd_v,
                   rows_v, wv2d_v, acc_sh, den_sh, sem):
    lanes = lax.broadcasted_iota(_i32, (16,), 0)

    def body(g, carry):
        sidx = src_v.at[g]
        didx = dst_v.at[g]
        c1 = pltpu.async_copy(h_ref.at[sidx], rows_v, sem)
        c2 = pltpu.async_copy(asrc_ref.at[sidx], avs_v, sem)
        c3 = pltpu.async_copy(adst_ref.at[didx], avd_v, sem)
        c1.wait()
        c2.wait()
        c3.wait()
        for j in range(8):
            sl = pl.ds(j * 16, 16)
            e = avs_v[sl] + avd_v[sl]
            e = jnp.maximum(e, e * 0.2)
            w16 = jnp.exp(e)

            def lane(l, c4, w16=w16, j=j):
                ws = jnp.sum(jnp.where(lanes == l, w16, 0.0))
                spl = ws * jnp.ones((16,), _f32)
                k = j * 16 + l
                wv2d_v[k, :] = spl
                for cc in range(4):
                    s2 = pl.ds(cc * 16, 16)
                    rows_v[k, s2] = rows_v[k, s2] * spl
                return c4

            lax.fori_loop(0, 16, lane, 0)
        pltpu.sync_copy(rows_v, acc_sh.at[didx], add=True)
        pltpu.sync_copy(wv2d_v, den_sh.at[didx], add=True)
        return carry
    return body


def _zero_stripes(sid, z64, z16, acc_sh, den_sh):
    zs = pl.multiple_of(sid * _SZ, 8)

    @pl.when(sid < 15)
    def _():
        pltpu.sync_copy(z64, acc_sh.at[pl.ds(zs, _SZ)])
        pltpu.sync_copy(z16, den_sh.at[pl.ds(zs, _SZ)])

    @pl.when(sid == 15)
    def _():
        pltpu.sync_copy(z64.at[pl.ds(0, _ZLAST)],
                        acc_sh.at[pl.ds(15 * _SZ, _ZLAST)])
        pltpu.sync_copy(z16.at[pl.ds(0, _ZLAST)],
                        den_sh.at[pl.ds(15 * _SZ, _ZLAST)])


def _out_stripes(sid, acc_sh, den_sh, num_slot, den_slot):
    os_ = pl.multiple_of(sid * _SZ, 8)

    @pl.when(sid < 15)
    def _():
        pltpu.sync_copy(acc_sh.at[pl.ds(os_, _SZ)],
                        num_slot.at[pl.ds(os_, _SZ)])
        pltpu.sync_copy(den_sh.at[pl.ds(os_, _SZ)],
                        den_slot.at[pl.ds(os_, _SZ)])

    @pl.when(sid == 15)
    def _():
        pltpu.sync_copy(acc_sh.at[pl.ds(15 * _SZ, _OLAST)],
                        num_slot.at[pl.ds(15 * _SZ, _OLAST)])
        pltpu.sync_copy(den_sh.at[pl.ds(15 * _SZ, _OLAST)],
                        den_slot.at[pl.ds(15 * _SZ, _OLAST)])


def _make_edge1():
    mesh = plsc.VectorSubcoreMesh(core_axis_name="c", subcore_axis_name="s")

    @functools.partial(
        pl.kernel, mesh=mesh,
        out_type=[jax.ShapeDtypeStruct((_HEADS, _N, _HID), _f32),
                  jax.ShapeDtypeStruct((_HEADS, _N, 16), _f32)],
        scratch_types=[
            pltpu.VMEM_SHARED((_NP16, _HID), _f32),
            pltpu.VMEM_SHARED((_NP16, 16), _f32),
            pltpu.VMEM((_GT1, 128), _i32),
            pltpu.VMEM((_GT1, 128), _i32),
            pltpu.VMEM((128,), _f32),
            pltpu.VMEM((128,), _f32),
            pltpu.VMEM((128, _HID), _f32),
            pltpu.VMEM((128, 16), _f32),
            pltpu.SemaphoreType.DMA,
        ],
        compiler_params=pltpu.CompilerParams(use_tc_tiling_on_sc=False, needs_layout_passes=False),
    )
    def edge1(src3, dst3, asrc, adst, z64, z16,
              h0, h1, h2, h3, h4, h5, h6, h7,
              num_out, den_out,
              acc_sh, den_sh, src_v, dst_v, avs_v, avd_v,
              rows_v, wv2d_v, sem):
        hs = (h0, h1, h2, h3, h4, h5, h6, h7)
        core = lax.axis_index("c")
        sid = lax.axis_index("s")
        gb = sid * _GT1
        pltpu.sync_copy(src3.at[pl.ds(gb, _GT1)], src_v)
        pltpu.sync_copy(dst3.at[pl.ds(gb, _GT1)], dst_v)
        for hp in range(_HEADS // 2):
            _zero_stripes(sid, z64, z16, acc_sh, den_sh)
            plsc.subcore_barrier()
            for c in range(2):
                head = c * (_HEADS // 2) + hp

                @pl.when(core == c)
                def _():
                    lax.fori_loop(
                        0, _GT1,
                        _sc_group_body(hs[head], asrc.at[head],
                                       adst.at[head], src_v, dst_v,
                                       avs_v, avd_v, rows_v, wv2d_v,
                                       acc_sh, den_sh, sem),
                        0)
            plsc.subcore_barrier()
            for c in range(2):
                head = c * (_HEADS // 2) + hp

                @pl.when(core == c)
                def _():
                    _out_stripes(sid, acc_sh, den_sh,
                                 num_out.at[head], den_out.at[head])
            plsc.subcore_barrier()

    return edge1


def _make_edge2():
    mesh = plsc.VectorSubcoreMesh(core_axis_name="c", subcore_axis_name="s")

    @functools.partial(
        pl.kernel, mesh=mesh,
        out_type=[jax.ShapeDtypeStruct((2, _N, _HID), _f32),
                  jax.ShapeDtypeStruct((2, _N, 16), _f32)],
        scratch_types=[
            pltpu.VMEM_SHARED((_NP16, _HID), _f32),
            pltpu.VMEM_SHARED((_NP16, 16), _f32),
            pltpu.VMEM((_GT2, 128), _i32),
            pltpu.VMEM((_GT2, 128), _i32),
            pltpu.VMEM((128,), _f32),
            pltpu.VMEM((128,), _f32),
            pltpu.VMEM((128, _HID), _f32),
            pltpu.VMEM((128, 16), _f32),
            pltpu.SemaphoreType.DMA,
        ],
        compiler_params=pltpu.CompilerParams(use_tc_tiling_on_sc=False, needs_layout_passes=False),
    )
    def edge2(src3, dst3, a2, z64, z16, hfeat,
              num_out, den_out,
              acc_sh, den_sh, src_v, dst_v, avs_v, avd_v,
              rows_v, wv2d_v, sem):
        core = lax.axis_index("c")
        sid = lax.axis_index("s")
        for c in range(2):
            @pl.when(core == c)
            def _():
                gb = c * (_GROUPS // 2) + sid * _GT2
                pltpu.sync_copy(src3.at[pl.ds(gb, _GT2)], src_v)
                pltpu.sync_copy(dst3.at[pl.ds(gb, _GT2)], dst_v)
        _zero_stripes(sid, z64, z16, acc_sh, den_sh)
        plsc.subcore_barrier()
        lax.fori_loop(
            0, _GT2,
            _sc_group_body(hfeat, a2.at[0], a2.at[1], src_v, dst_v,
                           avs_v, avd_v, rows_v, wv2d_v,
                           acc_sh, den_sh, sem),
            0)
        plsc.subcore_barrier()
        for c in range(2):
            @pl.when(core == c)
            def _():
                _out_stripes(sid, acc_sh, den_sh,
                             num_out.at[c], den_out.at[c])

    return edge2


def _finish1_body(num_ref, den_ref, b1_ref, w2_ref, as2_ref, ad2_ref,
                  h2_ref, a2s_ref, a2d_ref):
    acc = jnp.zeros((_BR, _HID), _f32)
    for h in range(_HEADS):
        seg = num_ref[h] / den_ref[h, :, 0:1]
        seg = jnp.maximum(seg + b1_ref[h:h + 1, :], 0.0)
        acc = acc + jnp.dot(seg, w2_ref[h], preferred_element_type=_f32)
    h2_ref[...] = acc
    a2s_ref[...] = jnp.sum(acc * as2_ref[...], axis=1, keepdims=True)
    a2d_ref[...] = jnp.sum(acc * ad2_ref[...], axis=1, keepdims=True)


def _finish1(num1, den1, b1r, W2r, att_src2, att_dst2):
    return pl.pallas_call(
        _finish1_body,
        grid=(_GRID,),
        in_specs=[
            pl.BlockSpec((_HEADS, _BR, _HID), lambda i: (0, i, 0)),
            pl.BlockSpec((_HEADS, _BR, 16), lambda i: (0, i, 0)),
            pl.BlockSpec((_HEADS, _HID), lambda i: (0, 0)),
            pl.BlockSpec((_HEADS, _HID, _HID), lambda i: (0, 0, 0)),
            pl.BlockSpec((1, _HID), lambda i: (0, 0)),
            pl.BlockSpec((1, _HID), lambda i: (0, 0)),
        ],
        out_specs=[
            pl.BlockSpec((_BR, _HID), lambda i: (i, 0)),
            pl.BlockSpec((_BR, 1), lambda i: (i, 0)),
            pl.BlockSpec((_BR, 1), lambda i: (i, 0)),
        ],
        out_shape=[jax.ShapeDtypeStruct((_N, _HID), _f32),
                   jax.ShapeDtypeStruct((_N, 1), _f32),
                   jax.ShapeDtypeStruct((_N, 1), _f32)],
    )(num1, den1, b1r, W2r, att_src2, att_dst2)


def _finish2_body(num_ref, den_ref, batch_ref, b2_ref, wfc_ref, out_ref):
    bid = batch_ref[...]
    oh = (lax.broadcasted_iota(_i32, (_G, _N), 0) == bid).astype(_f32)
    hsum = num_ref[0] + num_ref[1]
    dsum = den_ref[0, :, 0:1] + den_ref[1, :, 0:1]
    h2o = jnp.maximum(hsum / dsum + b2_ref[0:1, :], 0.0)
    ps = lax.dot_general(oh, h2o, (((1,), (0,)), ((), ())),
                         preferred_element_type=_f32)
    cnt = jnp.sum(oh, axis=1, keepdims=True)
    pooled = ps / jnp.maximum(cnt, 1.0)
    out_ref[...] = jnp.dot(pooled, wfc_ref[...], preferred_element_type=_f32)


def _finish2(num2, den2, batch2, b2r, Wfcp):
    return pl.pallas_call(
        _finish2_body,
        out_shape=jax.ShapeDtypeStruct((_G, 128), _f32),
    )(num2, den2, batch2, b2r, Wfcp)


_edge1_call = _make_edge1()
_edge2_call = _make_edge2()


def kernel(x, edge_index, batch, W1, att_src1, att_dst1, b1,
           W2, att_src2, att_dst2, b2, Wfc, bfc):
    loop = jnp.arange(_N, dtype=_i32)
    src = jnp.concatenate([edge_index[0].astype(_i32), loop])
    dst = jnp.concatenate([edge_index[1].astype(_i32), loop])
    padn = _EPAD - (_E + _N)
    src = jnp.concatenate([src, jnp.zeros((padn,), _i32)])
    dst = jnp.concatenate([dst, jnp.full((padn,), _N, _i32)])
    src3 = src.reshape(_GROUPS, 128)
    dst3 = dst.reshape(_GROUPS, 128)
    z64 = jnp.zeros((_SZ, _HID), _f32)
    z16 = jnp.zeros((_SZ, 16), _f32)

    d1 = _dense1(x, W1, att_src1, att_dst1)
    hs = d1[:_HEADS]
    asrc1 = jnp.concatenate([r.reshape(1, _N) for r in d1[_HEADS:2 * _HEADS]])
    adst1 = jnp.concatenate([r.reshape(1, _N) for r in d1[2 * _HEADS:]])

    num1, den1 = _edge1_call(src3, dst3, asrc1, adst1, z64, z16, *hs)

    h2f, a2s, a2d = _finish1(num1, den1, b1.reshape(_HEADS, _HID),
                             W2.reshape(_HEADS, _HID, _HID),
                             att_src2, att_dst2)
    a2 = jnp.concatenate([a2s.reshape(1, _N), a2d.reshape(1, _N)])

    num2, den2 = _edge2_call(src3, dst3, a2, z64, z16, h2f)

    batch2 = batch.astype(_i32).reshape(1, _N)
    b2r = b2.reshape(1, _HID)
    Wfcp = jnp.concatenate([Wfc, jnp.zeros((_HID, 127), _f32)], axis=1)
    fcout = _finish2(num2, den2, batch2, b2r, Wfcp)
    return fcout[:, :1] + bfc
